# Initial kernel scaffold; baseline (speedup 1.0000x reference)
import jax
import jax.numpy as jnp
from jax.experimental import pallas as pl


def kernel(messages, edge_index, num_nodes):
    raise NotImplementedError("write your pallas kernel here")


# speedup vs baseline: 3.8403x; 3.8403x over previous
import jax
import jax.numpy as jnp
from jax import lax
from jax.experimental import pallas as pl
from jax.experimental.pallas import tpu as pltpu
from jax.experimental.pallas import tpu_sc as plsc

_NC, _NS = 2, 16
_E = 320000
_D = 128
_N = 10000
_HALF = _N // _NC
_EPT = _E // _NS
_B = 128
_NFULL = _EPT // _B
_TAIL = _EPT - _NFULL * _B
_NBLK = _NFULL + 1
_ACC_ROWS = 5120
_ZPT = _ACC_ROWS // _NS
_OPT = 312


def _sc_body(msg_hbm, idx_hbm, out_hbm, idx_v, buf0, buf1, zbuf, acc, s0, s1):
    c = lax.axis_index("c")
    s = lax.axis_index("s")
    row0 = s * _EPT

    pltpu.sync_copy(idx_hbm.at[c, s], idx_v)

    def _z(r, carry):
        for k in range(_D // 16):
            zbuf[r, pl.ds(k * 16, 16)] = jnp.zeros((16,), jnp.float32)
        return carry

    lax.fori_loop(0, 128, _z, 0)
    pltpu.sync_copy(zbuf, acc.at[pl.ds(s * _ZPT, 128)])
    pltpu.sync_copy(zbuf, acc.at[pl.ds(s * _ZPT + 128, 128)])
    pltpu.sync_copy(zbuf.at[pl.ds(0, 64)], acc.at[pl.ds(s * _ZPT + 256, 64)])
    plsc.subcore_barrier()

    pltpu.async_copy(msg_hbm.at[pl.ds(row0, _B)], buf0, s0)
    pltpu.async_copy(msg_hbm.at[pl.ds(row0 + _B, _B)], buf1, s1)

    def _pair(i, carry):
        j0 = i * 2
        j1 = j0 + 1
        pltpu.make_async_copy(msg_hbm.at[pl.ds(row0 + j0 * _B, _B)], buf0, s0).wait()
        pltpu.sync_copy(buf0, acc.at[idx_v.at[j0]], add=True)

        @pl.when(j0 + 2 < _NFULL)
        def _():
            pltpu.async_copy(msg_hbm.at[pl.ds(row0 + (j0 + 2) * _B, _B)], buf0, s0)

        pltpu.make_async_copy(msg_hbm.at[pl.ds(row0 + j1 * _B, _B)], buf1, s1).wait()
        pltpu.sync_copy(buf1, acc.at[idx_v.at[j1]], add=True)

        @pl.when(j1 + 2 < _NFULL)
        def _():
            pltpu.async_copy(msg_hbm.at[pl.ds(row0 + (j1 + 2) * _B, _B)], buf1, s1)

        return carry

    lax.fori_loop(0, _NFULL // 2, _pair, 0)

    pltpu.sync_copy(msg_hbm.at[pl.ds(row0 + _NFULL * _B, _TAIL)],
                    buf0.at[pl.ds(0, _TAIL)])
    pltpu.sync_copy(buf0, acc.at[idx_v.at[_NFULL]], add=True)

    plsc.subcore_barrier()

    pltpu.sync_copy(acc.at[pl.ds(s * _OPT, _OPT)],
                    out_hbm.at[pl.ds(c * _HALF + s * _OPT, _OPT)])

    @pl.when(s == _NS - 1)
    def _():
        pltpu.sync_copy(acc.at[pl.ds(_NS * _OPT, _HALF - _NS * _OPT)],
                        out_hbm.at[pl.ds(c * _HALF + _NS * _OPT,
                                         _HALF - _NS * _OPT)])


@jax.jit
def _run(messages, dst):
    dstp = jnp.concatenate(
        [dst.reshape(_NS, _EPT),
         jnp.full((_NS, _NBLK * _B - _EPT), jnp.int32(1 << 30))], axis=1)
    halves = []
    for core in range(_NC):
        local = dstp - jnp.int32(core * _HALF)
        ok = (local >= 0) & (local < _HALF)
        halves.append(jnp.where(ok, local, jnp.int32(_HALF)))
    idx = jnp.stack(halves).reshape(_NC, _NS, _NBLK, _B)

    mesh = plsc.VectorSubcoreMesh(core_axis_name="c", subcore_axis_name="s",
                                  num_cores=_NC, num_subcores=_NS)
    return pl.kernel(
        _sc_body,
        out_type=jax.ShapeDtypeStruct((_N, _D), jnp.float32),
        mesh=mesh,
        scratch_types=[
            pltpu.VMEM((_NBLK, _B), jnp.int32),
            pltpu.VMEM((_B, _D), jnp.float32),
            pltpu.VMEM((_B, _D), jnp.float32),
            pltpu.VMEM((128, _D), jnp.float32),
            pltpu.VMEM_SHARED((_ACC_ROWS, _D), jnp.float32),
            pltpu.SemaphoreType.DMA,
            pltpu.SemaphoreType.DMA,
        ],
    )(messages, idx)


def kernel(messages, edge_index, num_nodes):
    return _run(messages, edge_index[1].astype(jnp.int32))
